# Initial kernel scaffold; baseline (speedup 1.0000x reference)
import jax
import jax.numpy as jnp
from jax.experimental import pallas as pl


def kernel(inputs):
    raise NotImplementedError("write your pallas kernel here")


# speedup vs baseline: 1.5970x; 1.5970x over previous
import functools

import jax
import jax.numpy as jnp
from jax import lax
from jax.experimental import pallas as pl
from jax.experimental.pallas import tpu as pltpu
from jax.experimental.pallas import tpu_sc as plsc

ROWS, N = 128, 32768
NC, NS = 2, 16
NW = NC * NS
RPW = ROWS // NW
LANES = 16
CHUNK = N // LANES
NBINS = 256
FMASK = 0x7FFFFFFF


def _sort_body(in_hbm, out_hbm, row_f, bufa, bufb, hist):
    wid = lax.axis_index("s") * NC + lax.axis_index("c")
    lane = lax.iota(jnp.int32, LANES)
    base_idx = lane * CHUNK
    ones = jnp.ones((LANES,), jnp.int32)
    zeros = jnp.zeros((LANES,), jnp.int32)

    def load_key_f32(i):
        x = plsc.load_gather(row_f, [base_idx + i])
        u = plsc.bitcast(x, jnp.int32)
        return jnp.where(u < 0, u, u ^ FMASK)

    def store_i32(dst, pos, key):
        plsc.store_scatter(dst, [pos], key)

    def store_f32(dst, pos, key):
        v = jnp.where(key < 0, key, key ^ FMASK)
        plsc.store_scatter(dst, [pos], plsc.bitcast(v, jnp.float32))

    def one_pass(load_key, store_val, dst, shift):
        def zero_body(j, _):
            hist[pl.ds(j * LANES, LANES)] = zeros
            return 0

        lax.fori_loop(0, NBINS, zero_body, 0)

        def hist_body(i, _):
            key = load_key(i)
            d = lax.shift_right_logical(key, shift) & 0xFF
            flat = (d << 4) | lane
            plsc.addupdate_scatter(hist, [flat], ones)
            return 0

        lax.fori_loop(0, CHUNK, hist_body, 0)

        def scan_body(j, carry):
            v = hist[pl.ds(j * LANES, LANES)]
            excl = plsc.cumsum(v) - v
            hist[pl.ds(j * LANES, LANES)] = excl + carry
            return carry + jnp.sum(v)

        lax.fori_loop(0, NBINS, scan_body, jnp.int32(0))

        def perm_body(i, _):
            key = load_key(i)
            d = lax.shift_right_logical(key, shift) & 0xFF
            flat = (d << 4) | lane
            pos = plsc.load_gather(hist, [flat])
            plsc.addupdate_scatter(hist, [flat], ones)
            store_val(dst, pos, key)
            return 0

        lax.fori_loop(0, CHUNK, perm_body, 0)

    for r in range(RPW):
        row = wid * RPW + r
        pltpu.sync_copy(in_hbm.at[row], row_f)
        one_pass(load_key_f32, store_i32, bufa, 0)
        one_pass(lambda i: plsc.load_gather(bufa, [base_idx + i]), store_i32, bufb, 8)
        one_pass(lambda i: plsc.load_gather(bufb, [base_idx + i]), store_i32, bufa, 16)
        one_pass(lambda i: plsc.load_gather(bufa, [base_idx + i]), store_f32, row_f, 24)
        pltpu.sync_copy(row_f, out_hbm.at[row])


@functools.partial(
    pl.kernel,
    out_type=jax.ShapeDtypeStruct((ROWS, N), jnp.float32),
    mesh=plsc.VectorSubcoreMesh(core_axis_name="c", subcore_axis_name="s"),
    scratch_types=[
        pltpu.VMEM((N,), jnp.float32),
        pltpu.VMEM((N,), jnp.int32),
        pltpu.VMEM((N,), jnp.int32),
        pltpu.VMEM((NBINS * LANES,), jnp.int32),
    ],
    compiler_params=pltpu.CompilerParams(needs_layout_passes=False),
)
def _sort_kernel(in_hbm, out_hbm, row_f, bufa, bufb, hist):
    _sort_body(in_hbm, out_hbm, row_f, bufa, bufb, hist)


def kernel(inputs):
    return _sort_kernel(inputs)
